# Initial kernel scaffold; baseline (speedup 1.0000x reference)
import jax
import jax.numpy as jnp
from jax.experimental import pallas as pl


def kernel(x, logits):
    raise NotImplementedError("write your pallas kernel here")


# speedup vs baseline: 1.0431x; 1.0431x over previous
import functools

import jax
import jax.numpy as jnp
from jax import lax
from jax.experimental import pallas as pl
from jax.experimental.pallas import tpu as pltpu
from jax.experimental.pallas import tpu_sc as plsc

_B, _S, _C = 16, 1024, 768
_K = 128


def _select_body(logits_ref, sel_ref):
    b = pl.program_id(0)
    lg = logits_ref[0]
    m = jnp.max(lg, axis=-1, keepdims=True)
    e = jnp.exp(lg - m)
    s = jnp.sum(e, axis=-1, keepdims=True)
    p = e / s
    vcol = jnp.max(p, axis=-1, keepdims=True)
    vrow = lax.transpose(vcol, (1, 0))
    ii = lax.broadcasted_iota(jnp.int32, (_S, _S), 0)
    jj = lax.broadcasted_iota(jnp.int32, (_S, _S), 1)
    vi = jnp.broadcast_to(vcol, (_S, _S))
    vj = jnp.broadcast_to(vrow, (_S, _S))
    win = (vj > vi) | ((vj == vi) & (jj < ii))
    rank = jnp.sum(win.astype(jnp.int32), axis=1, keepdims=True)
    rr = lax.broadcasted_iota(jnp.int32, (_S, _K), 1)
    i2 = lax.broadcasted_iota(jnp.int32, (_S, _K), 0)
    onehot = jnp.broadcast_to(rank, (_S, _K)) == rr
    sel = jnp.sum(jnp.where(onehot, i2 + b * _S, 0), axis=0, keepdims=True)
    sel_ref[0] = jnp.broadcast_to(sel, (8, _K))


def _select(logits):
    nc = logits.shape[-1]
    return pl.pallas_call(
        _select_body,
        grid=(_B,),
        in_specs=[pl.BlockSpec((1, _S, nc), lambda b: (b, 0, 0))],
        out_specs=pl.BlockSpec((1, 8, _K), lambda b: (b, 0, 0)),
        out_shape=jax.ShapeDtypeStruct((_B, 8, _K), jnp.int32),
    )(logits)


def _gather(xflat, idx):
    info = plsc.get_sparse_core_info()
    nw = info.num_cores * info.num_subcores
    n = idx.shape[0]
    bpw = n // nw
    mesh = plsc.VectorSubcoreMesh(core_axis_name="c", subcore_axis_name="s")

    @functools.partial(
        pl.kernel, mesh=mesh,
        out_type=jax.ShapeDtypeStruct((n, _C), jnp.float32),
        scratch_types=[
            pltpu.VMEM((bpw,), jnp.int32),
            pltpu.VMEM((bpw, _C), jnp.float32),
            pltpu.SemaphoreType.DMA,
        ],
    )
    def k(table_hbm, idx_hbm, out_hbm, idx_v, rows_v, sem):
        wid = lax.axis_index("s") * info.num_cores + lax.axis_index("c")
        base = wid * bpw
        pltpu.sync_copy(idx_hbm.at[pl.ds(base, bpw)], idx_v)
        pltpu.async_copy(table_hbm.at[idx_v], rows_v, sem).wait()
        pltpu.sync_copy(rows_v, out_hbm.at[pl.ds(base, bpw)])

    return k(xflat, idx)


def kernel(x, logits):
    sel = _select(logits)
    idx = sel[:, 0, :].reshape(_B * _K)
    rows = _gather(x.reshape(_B * _S, _C), idx)
    return rows.reshape(_B, _K, _C)
